# baseline (speedup 1.0000x reference)
import functools
import math

import jax
import jax.numpy as jnp
from jax import lax
from jax.experimental import pallas as pl
from jax.experimental.pallas import tpu as pltpu
from jax.experimental.pallas import tpu_sc as plsc

N = 10000
E = 160000
C = 4
D = 128
NG = C + 1
NP = 10240
NC = 2
NS = 16
NW = NC * NS
CH = 128
NCH = 40
ESUB = NCH * CH
EP = NW * ESUB


def _sc_degrees(dst_flat):
    mesh = plsc.VectorSubcoreMesh(core_axis_name="c", subcore_axis_name="s")

    @functools.partial(
        pl.kernel,
        out_type=jax.ShapeDtypeStruct((NW * NG * NP,), jnp.float32),
        mesh=mesh,
        scratch_types=[
            pltpu.VMEM((ESUB,), jnp.int32),
            pltpu.VMEM((NP,), jnp.float32),
        ],
        compiler_params=pltpu.CompilerParams(needs_layout_passes=False),
    )
    def k(dst_hbm, degp_hbm, idx_v, acc):
        cc = lax.axis_index("c")
        ss = lax.axis_index("s")
        wid = ss * NC + cc
        base = wid * ESUB
        ones = jnp.full((16,), 1.0, jnp.float32)
        zeros16 = jnp.zeros((16,), jnp.float32)
        for g in range(NG):
            def zero_body(i, carry):
                acc[pl.ds(i * 16, 16)] = zeros16
                return carry
            lax.fori_loop(0, NP // 16, zero_body, 0)
            pltpu.sync_copy(dst_hbm.at[pl.ds(g * EP + base, ESUB)], idx_v)

            def body(i, carry):
                idx = idx_v[pl.ds(i * 16, 16)]
                plsc.addupdate_scatter(acc, [idx], ones)
                return carry
            lax.fori_loop(0, ESUB // 16, body, 0)
            pltpu.sync_copy(acc, degp_hbm.at[pl.ds((wid * NG + g) * NP, NP)])

    return k(dst_flat).reshape(NW, NG, NP)


def _sc_scatter(hs, src_flat, dst_flat):
    RZ = 64
    RPS = NP // NS
    mesh = plsc.VectorSubcoreMesh(core_axis_name="c", subcore_axis_name="s")

    @functools.partial(
        pl.kernel,
        out_type=jax.ShapeDtypeStruct((NC, NG, NP, D), jnp.float32),
        mesh=mesh,
        scratch_types=[
            pltpu.VMEM((CH,), jnp.int32),
            pltpu.VMEM((CH,), jnp.int32),
            pltpu.VMEM((CH,), jnp.int32),
            pltpu.VMEM((CH,), jnp.int32),
            pltpu.VMEM((CH, D), jnp.float32),
            pltpu.VMEM((CH, D), jnp.float32),
            pltpu.VMEM((RZ, D), jnp.float32),
            pltpu.VMEM_SHARED((NP, D), jnp.float32),
            pltpu.SemaphoreType.DMA,
            pltpu.SemaphoreType.DMA,
        ],
    )
    def k(hs_hbm, src_hbm, dst_hbm, part_hbm,
          isrc0, isrc1, idst0, idst1, rows0, rows1, zbuf, accsh, sem0, sem1):
        cc = lax.axis_index("c")
        ss = lax.axis_index("s")
        wid = cc * NS + ss
        row0 = ss * RPS
        zeros16 = jnp.zeros((16,), jnp.float32)

        def zb(i, carry):
            r = i // (D // 16)
            col = (i % (D // 16)) * 16
            zbuf[r, pl.ds(col, 16)] = zeros16
            return carry
        lax.fori_loop(0, RZ * (D // 16), zb, 0)

        for g in range(NG):
            goff = g * EP + wid * ESUB
            for t in range(RPS // RZ):
                pltpu.sync_copy(zbuf, accsh.at[pl.ds(row0 + t * RZ, RZ)])
            plsc.subcore_barrier()

            def body(i, carry):
                eoff = goff + i * CH
                pltpu.sync_copy(src_hbm.at[pl.ds(eoff, CH)], isrc0)
                pltpu.sync_copy(dst_hbm.at[pl.ds(eoff, CH)], idst0)
                pltpu.async_copy(hs_hbm.at[g].at[isrc0], rows0, sem0).wait()
                pltpu.sync_copy(rows0, accsh.at[idst0], add=True)
                return carry
            lax.fori_loop(0, NCH, body, 0)
            plsc.subcore_barrier()
            pltpu.sync_copy(accsh.at[pl.ds(row0, RPS)],
                            part_hbm.at[cc, g, pl.ds(row0, RPS)])

    return k(hs, src_flat, dst_flat)


def _tc_dinv(degp):
    def body(degp_ref, dinv_ref):
        deg = jnp.sum(degp_ref[...], axis=0) + 1.0
        dinv_ref[...] = lax.rsqrt(jnp.maximum(deg, 1.0))

    return pl.pallas_call(
        body, out_shape=jax.ShapeDtypeStruct((NG, NP), jnp.float32)
    )(degp)


def _tc_prep(x_all, W, dinv):
    BN = 512
    grid = (NG, NP // BN)

    def body(x_ref, w_ref, dinv_ref, hs_ref):
        h = jnp.dot(x_ref[0], w_ref[...], preferred_element_type=jnp.float32)
        hs_ref[0] = h * dinv_ref[0]

    return pl.pallas_call(
        body,
        grid=grid,
        in_specs=[
            pl.BlockSpec((1, BN, D), lambda g, i: (g, i, 0)),
            pl.BlockSpec((D, D), lambda g, i: (0, 0)),
            pl.BlockSpec((1, BN, 1), lambda g, i: (g, i, 0)),
        ],
        out_specs=pl.BlockSpec((1, BN, D), lambda g, i: (g, i, 0)),
        out_shape=jax.ShapeDtypeStruct((NG, NP, D), jnp.float32),
    )(x_all, W, dinv[:, :, None])


def _tc_post(part, hs, dinv, b, label_emb, Wq, bq, Wk, bk, Wv, bv, do_relu):
    BN = 256
    grid = (NP // BN,)
    scale = 1.0 / math.sqrt(D)

    def body(part_ref, hs_ref, dinv_ref, b_ref, le_ref,
             wq_ref, bq_ref, wk_ref, bk_ref, wv_ref, bv_ref,
             gfe_ref, ofe_ref):
        dv = dinv_ref[...]
        conv = (part_ref[0] + part_ref[1] + hs_ref[...]) * dv[:, :, None] \
            + b_ref[...][None, None, :]
        lq = jnp.dot(le_ref[...], wq_ref[...],
                     preferred_element_type=jnp.float32) + bq_ref[...][None, :]
        ks = [jnp.dot(conv[j], wk_ref[...],
                      preferred_element_type=jnp.float32) + bk_ref[...][None, :]
              for j in range(C)]
        vs = [jnp.dot(conv[j], wv_ref[...],
                      preferred_element_type=jnp.float32) + bv_ref[...][None, :]
              for j in range(C)]
        s = jnp.stack(
            [jnp.dot(kj, lq.T, preferred_element_type=jnp.float32)
             for kj in ks], axis=-1) * scale
        s = s - jnp.max(s, axis=-1, keepdims=True)
        p = jnp.exp(s)
        p = p / jnp.sum(p, axis=-1, keepdims=True)
        for a in range(C):
            out_a = p[:, a, 0][:, None] * vs[0]
            for jb in range(1, C):
                out_a = out_a + p[:, a, jb][:, None] * vs[jb]
            if do_relu:
                out_a = jnp.maximum(out_a, 0.0)
            gfe_ref[a] = out_a
        ofe = conv[C]
        if do_relu:
            ofe = jnp.maximum(ofe, 0.0)
        ofe_ref[...] = ofe

    return pl.pallas_call(
        body,
        grid=grid,
        in_specs=[
            pl.BlockSpec((NC, NG, BN, D), lambda i: (0, 0, i, 0)),
            pl.BlockSpec((NG, BN, D), lambda i: (0, i, 0)),
            pl.BlockSpec((NG, BN), lambda i: (0, i)),
            pl.BlockSpec((D,), lambda i: (0,)),
            pl.BlockSpec((C, D), lambda i: (0, 0)),
            pl.BlockSpec((D, D), lambda i: (0, 0)),
            pl.BlockSpec((D,), lambda i: (0,)),
            pl.BlockSpec((D, D), lambda i: (0, 0)),
            pl.BlockSpec((D,), lambda i: (0,)),
            pl.BlockSpec((D, D), lambda i: (0, 0)),
            pl.BlockSpec((D,), lambda i: (0,)),
        ],
        out_specs=[
            pl.BlockSpec((C, BN, D), lambda i: (0, i, 0)),
            pl.BlockSpec((BN, D), lambda i: (i, 0)),
        ],
        out_shape=[
            jax.ShapeDtypeStruct((C, NP, D), jnp.float32),
            jax.ShapeDtypeStruct((NP, D), jnp.float32),
        ],
    )(part, hs, dinv, b, label_emb, Wq, bq, Wk, bk, Wv, bv)


def kernel(graph_feat_emb, ori_feat_emb, label_emb, edge_index_label,
           edge_index_ori, W0, b0, W1, b1, Wq, bq, Wk, bk, Wv, bv):
    pad = NP - N
    gfe0 = jnp.pad(graph_feat_emb, ((0, 0), (0, pad), (0, 0)))
    ofe0 = jnp.pad(ori_feat_emb, ((0, pad), (0, 0)))
    src_all = jnp.concatenate(
        [edge_index_label[:, 0, :], edge_index_ori[0][None]], axis=0)
    dst_all = jnp.concatenate(
        [edge_index_label[:, 1, :], edge_index_ori[1][None]], axis=0)
    src_flat = jnp.pad(src_all, ((0, 0), (0, EP - E)),
                       constant_values=N).reshape(-1)
    dst_flat = jnp.pad(dst_all, ((0, 0), (0, EP - E)),
                       constant_values=N).reshape(-1)

    degp = _sc_degrees(dst_flat)
    dinv = _tc_dinv(degp)

    x = jnp.concatenate([gfe0, ofe0[None]], axis=0)
    gfe = None
    ofe = None
    for li, (W, b) in enumerate(((W0, b0), (W1, b1))):
        hs = _tc_prep(x, W, dinv)
        part = _sc_scatter(hs, src_flat, dst_flat)
        gfe, ofe = _tc_post(part, hs, dinv, b, label_emb,
                            Wq, bq, Wk, bk, Wv, bv, do_relu=(li == 0))
        if li == 0:
            x = jnp.concatenate([gfe, ofe[None]], axis=0)

    return gfe[:, :N, :], ofe[:N, :]


# speedup vs baseline: 1.0098x; 1.0098x over previous
import functools
import math

import jax
import jax.numpy as jnp
from jax import lax
from jax.experimental import pallas as pl
from jax.experimental.pallas import tpu as pltpu
from jax.experimental.pallas import tpu_sc as plsc

N = 10000
E = 160000
C = 4
D = 128
NG = C + 1
NP = 10240
NC = 2
NS = 16
NW = NC * NS
CH = 128
NCH = 40
ESUB = NCH * CH
EP = NW * ESUB


def _sc_degrees(dst_flat):
    mesh = plsc.VectorSubcoreMesh(core_axis_name="c", subcore_axis_name="s")

    @functools.partial(
        pl.kernel,
        out_type=jax.ShapeDtypeStruct((NW * NG * NP,), jnp.float32),
        mesh=mesh,
        scratch_types=[
            pltpu.VMEM((ESUB,), jnp.int32),
            pltpu.VMEM((NP,), jnp.float32),
        ],
        compiler_params=pltpu.CompilerParams(needs_layout_passes=False),
    )
    def k(dst_hbm, degp_hbm, idx_v, acc):
        cc = lax.axis_index("c")
        ss = lax.axis_index("s")
        wid = ss * NC + cc
        base = wid * ESUB
        nvec = jnp.full((16,), N, jnp.int32)
        ones = jnp.full((16,), 1.0, jnp.float32)
        zeros16 = jnp.zeros((16,), jnp.float32)
        for g in range(NG):
            def zero_body(i, carry):
                acc[pl.ds(i * 16, 16)] = zeros16
                return carry
            lax.fori_loop(0, NP // 16, zero_body, 0)
            pltpu.sync_copy(dst_hbm.at[pl.ds(g * EP + base, ESUB)], idx_v)

            def body(i, carry):
                idx = idx_v[pl.ds(i * 16, 16)]
                vals = jnp.where(idx < nvec, ones, zeros16)
                plsc.addupdate_scatter(acc, [idx], vals)
                return carry
            lax.fori_loop(0, ESUB // 16, body, 0)
            pltpu.sync_copy(acc, degp_hbm.at[pl.ds((wid * NG + g) * NP, NP)])

    return k(dst_flat).reshape(NW, NG, NP)


def _sc_scatter(hs, src_flat, dst_flat):
    RZ = 64
    RPS = NP // NS
    mesh = plsc.VectorSubcoreMesh(core_axis_name="c", subcore_axis_name="s")

    @functools.partial(
        pl.kernel,
        out_type=jax.ShapeDtypeStruct((NC, NG, NP, D), jnp.float32),
        mesh=mesh,
        scratch_types=[
            pltpu.VMEM((CH,), jnp.int32),
            pltpu.VMEM((CH,), jnp.int32),
            pltpu.VMEM((CH,), jnp.int32),
            pltpu.VMEM((CH,), jnp.int32),
            pltpu.VMEM((CH, D), jnp.float32),
            pltpu.VMEM((CH, D), jnp.float32),
            pltpu.VMEM((RZ, D), jnp.float32),
            pltpu.VMEM_SHARED((NP, D), jnp.float32),
            pltpu.SemaphoreType.DMA,
            pltpu.SemaphoreType.DMA,
        ],
    )
    def k(hs_hbm, src_hbm, dst_hbm, part_hbm,
          isrc0, isrc1, idst0, idst1, rows0, rows1, zbuf, accsh, sem0, sem1):
        cc = lax.axis_index("c")
        ss = lax.axis_index("s")
        wid = cc * NS + ss
        row0 = ss * RPS
        zeros16 = jnp.zeros((16,), jnp.float32)

        def zb(i, carry):
            r = i // (D // 16)
            col = (i % (D // 16)) * 16
            zbuf[r, pl.ds(col, 16)] = zeros16
            return carry
        lax.fori_loop(0, RZ * (D // 16), zb, 0)

        for g in range(NG):
            goff = g * EP + wid * ESUB
            for t in range(RPS // RZ):
                pltpu.sync_copy(zbuf, accsh.at[pl.ds(row0 + t * RZ, RZ)])
            plsc.subcore_barrier()

            def body(i, carry):
                eoff = goff + i * CH
                pltpu.sync_copy(src_hbm.at[pl.ds(eoff, CH)], isrc0)
                pltpu.sync_copy(dst_hbm.at[pl.ds(eoff, CH)], idst0)
                pltpu.async_copy(hs_hbm.at[g].at[isrc0], rows0, sem0).wait()
                pltpu.sync_copy(rows0, accsh.at[idst0], add=True)
                return carry
            lax.fori_loop(0, NCH, body, 0)
            plsc.subcore_barrier()
            pltpu.sync_copy(accsh.at[pl.ds(row0, RPS)],
                            part_hbm.at[cc, g, pl.ds(row0, RPS)])

    return k(hs, src_flat, dst_flat)


def _tc_dinv(degp):
    def body(degp_ref, dinv_ref):
        deg = jnp.sum(degp_ref[...], axis=0) + 1.0
        dinv_ref[...] = lax.rsqrt(jnp.maximum(deg, 1.0))

    return pl.pallas_call(
        body, out_shape=jax.ShapeDtypeStruct((NG, NP), jnp.float32)
    )(degp)


def _tc_prep(x_all, W, dinv):
    BN = 512
    grid = (NG, NP // BN)

    def body(x_ref, w_ref, dinv_ref, hs_ref):
        h = jnp.dot(x_ref[0], w_ref[...], preferred_element_type=jnp.float32)
        i = pl.program_id(1)
        rows = i * BN + lax.broadcasted_iota(jnp.int32, (BN, 1), 0)
        hs_ref[0] = jnp.where(rows < N, h * dinv_ref[0], 0.0)

    return pl.pallas_call(
        body,
        grid=grid,
        in_specs=[
            pl.BlockSpec((1, BN, D), lambda g, i: (g, i, 0)),
            pl.BlockSpec((D, D), lambda g, i: (0, 0)),
            pl.BlockSpec((1, BN, 1), lambda g, i: (g, i, 0)),
        ],
        out_specs=pl.BlockSpec((1, BN, D), lambda g, i: (g, i, 0)),
        out_shape=jax.ShapeDtypeStruct((NG, NP, D), jnp.float32),
    )(x_all, W, dinv[:, :, None])


def _tc_post(part, hs, dinv, b, label_emb, Wq, bq, Wk, bk, Wv, bv, do_relu):
    BN = 256
    grid = (NP // BN,)
    scale = 1.0 / math.sqrt(D)

    def body(part_ref, hs_ref, dinv_ref, b_ref, le_ref,
             wq_ref, bq_ref, wk_ref, bk_ref, wv_ref, bv_ref,
             gfe_ref, ofe_ref):
        dv = dinv_ref[...]
        conv = (part_ref[0] + part_ref[1] + hs_ref[...]) * dv[:, :, None] \
            + b_ref[...][None, None, :]
        lq = jnp.dot(le_ref[...], wq_ref[...],
                     preferred_element_type=jnp.float32) + bq_ref[...][None, :]
        ks = [jnp.dot(conv[j], wk_ref[...],
                      preferred_element_type=jnp.float32) + bk_ref[...][None, :]
              for j in range(C)]
        vs = [jnp.dot(conv[j], wv_ref[...],
                      preferred_element_type=jnp.float32) + bv_ref[...][None, :]
              for j in range(C)]
        s = jnp.stack(
            [jnp.dot(kj, lq.T, preferred_element_type=jnp.float32)
             for kj in ks], axis=-1) * scale
        s = s - jnp.max(s, axis=-1, keepdims=True)
        p = jnp.exp(s)
        p = p / jnp.sum(p, axis=-1, keepdims=True)
        for a in range(C):
            out_a = p[:, a, 0][:, None] * vs[0]
            for jb in range(1, C):
                out_a = out_a + p[:, a, jb][:, None] * vs[jb]
            if do_relu:
                out_a = jnp.maximum(out_a, 0.0)
            gfe_ref[a] = out_a
        ofe = conv[C]
        if do_relu:
            ofe = jnp.maximum(ofe, 0.0)
        ofe_ref[...] = ofe

    return pl.pallas_call(
        body,
        grid=grid,
        in_specs=[
            pl.BlockSpec((NC, NG, BN, D), lambda i: (0, 0, i, 0)),
            pl.BlockSpec((NG, BN, D), lambda i: (0, i, 0)),
            pl.BlockSpec((NG, BN), lambda i: (0, i)),
            pl.BlockSpec((D,), lambda i: (0,)),
            pl.BlockSpec((C, D), lambda i: (0, 0)),
            pl.BlockSpec((D, D), lambda i: (0, 0)),
            pl.BlockSpec((D,), lambda i: (0,)),
            pl.BlockSpec((D, D), lambda i: (0, 0)),
            pl.BlockSpec((D,), lambda i: (0,)),
            pl.BlockSpec((D, D), lambda i: (0, 0)),
            pl.BlockSpec((D,), lambda i: (0,)),
        ],
        out_specs=[
            pl.BlockSpec((C, BN, D), lambda i: (0, i, 0)),
            pl.BlockSpec((BN, D), lambda i: (i, 0)),
        ],
        out_shape=[
            jax.ShapeDtypeStruct((C, NP, D), jnp.float32),
            jax.ShapeDtypeStruct((NP, D), jnp.float32),
        ],
    )(part, hs, dinv, b, label_emb, Wq, bq, Wk, bk, Wv, bv)


def kernel(graph_feat_emb, ori_feat_emb, label_emb, edge_index_label,
           edge_index_ori, W0, b0, W1, b1, Wq, bq, Wk, bk, Wv, bv):
    pad = NP - N
    gfe0 = jnp.pad(graph_feat_emb, ((0, 0), (0, pad), (0, 0)))
    ofe0 = jnp.pad(ori_feat_emb, ((0, pad), (0, 0)))
    src_all = jnp.concatenate(
        [edge_index_label[:, 0, :], edge_index_ori[0][None]], axis=0)
    dst_all = jnp.concatenate(
        [edge_index_label[:, 1, :], edge_index_ori[1][None]], axis=0)
    src_flat = jnp.pad(src_all, ((0, 0), (0, EP - E)),
                       constant_values=N).reshape(-1)
    spread = (jnp.arange(EP - E, dtype=jnp.int32) % N)[None, :]
    dst_flat = jnp.concatenate(
        [dst_all, jnp.broadcast_to(spread, (NG, EP - E))], axis=1).reshape(-1)
    dstd_flat = jnp.pad(dst_all, ((0, 0), (0, EP - E)),
                        constant_values=N).reshape(-1)

    degp = _sc_degrees(dstd_flat)
    dinv = _tc_dinv(degp)

    x = jnp.concatenate([gfe0, ofe0[None]], axis=0)
    gfe = None
    ofe = None
    for li, (W, b) in enumerate(((W0, b0), (W1, b1))):
        hs = _tc_prep(x, W, dinv)
        part = _sc_scatter(hs, src_flat, dst_flat)
        gfe, ofe = _tc_post(part, hs, dinv, b, label_emb,
                            Wq, bq, Wk, bk, Wv, bv, do_relu=(li == 0))
        if li == 0:
            x = jnp.concatenate([gfe, ofe[None]], axis=0)

    return gfe[:, :N, :], ofe[:N, :]


# speedup vs baseline: 1.8908x; 1.8725x over previous
import functools
import math

import jax
import jax.numpy as jnp
from jax import lax
from jax.experimental import pallas as pl
from jax.experimental.pallas import tpu as pltpu
from jax.experimental.pallas import tpu_sc as plsc

N = 10000
E = 160000
C = 4
D = 128
NG = C + 1
NP = 10240
NC = 2
NS = 16
NW = NC * NS


def _sc_degrees(dst_all):
    EW = E // NW
    n_full = EW // 16
    rem = EW - n_full * 16
    mesh = plsc.VectorSubcoreMesh(core_axis_name="c", subcore_axis_name="s")

    @functools.partial(
        pl.kernel,
        out_type=jax.ShapeDtypeStruct((NW * NG * NP,), jnp.float32),
        mesh=mesh,
        scratch_types=[
            pltpu.VMEM((EW + 16,), jnp.int32),
            pltpu.VMEM((NP,), jnp.float32),
        ],
        compiler_params=pltpu.CompilerParams(needs_layout_passes=False),
    )
    def k(dst_hbm, degp_hbm, idx_v, acc):
        cc = lax.axis_index("c")
        ss = lax.axis_index("s")
        wid = ss * NC + cc
        base = wid * EW
        ones = jnp.full((16,), 1.0, jnp.float32)
        zeros16 = jnp.zeros((16,), jnp.float32)
        for g in range(NG):
            def zero_body(i, carry):
                acc[pl.ds(i * 16, 16)] = zeros16
                return carry
            lax.fori_loop(0, NP // 16, zero_body, 0)
            idx_v[pl.ds(n_full * 16, 16)] = jnp.zeros((16,), jnp.int32)
            pltpu.sync_copy(dst_hbm.at[pl.ds(g * E + base, EW)],
                            idx_v.at[pl.ds(0, EW)])

            def body(i, carry):
                idx = idx_v[pl.ds(i * 16, 16)]
                plsc.addupdate_scatter(acc, [idx], ones)
                return carry
            lax.fori_loop(0, n_full, body, 0)
            if rem:
                idx = idx_v[pl.ds(n_full * 16, 16)]
                mask = lax.iota(jnp.int32, 16) < rem
                plsc.addupdate_scatter(acc, [idx], ones, mask=mask)
            pltpu.sync_copy(acc, degp_hbm.at[pl.ds((wid * NG + g) * NP, NP)])

    return k(dst_all.reshape(-1)).reshape(NW, NG, NP)


def _sc_scatter(hs, src_all, dst_all):
    EH = E // NC
    EWS = EH // NS
    CH = 128
    n_full = EWS // CH
    rem = EWS - n_full * CH
    RZ = 128
    RPS = NP // NS
    mesh = plsc.VectorSubcoreMesh(core_axis_name="c", subcore_axis_name="s")

    @functools.partial(
        pl.kernel,
        out_type=jax.ShapeDtypeStruct((NC, NG, NP, D), jnp.float32),
        mesh=mesh,
        scratch_types=[
            pltpu.VMEM((CH,), jnp.int32),
            pltpu.VMEM((CH,), jnp.int32),
            pltpu.VMEM((CH, D), jnp.float32),
            pltpu.VMEM((rem,), jnp.int32),
            pltpu.VMEM((rem,), jnp.int32),
            pltpu.VMEM((rem, D), jnp.float32),
            pltpu.VMEM((RZ, D), jnp.float32),
            pltpu.VMEM_SHARED((NP, D), jnp.float32),
            pltpu.SemaphoreType.DMA,
        ],
    )
    def k(hs_hbm, src_hbm, dst_hbm, part_hbm,
          isrc, idst, rows, isrc_t, idst_t, rows_t, zbuf, accsh, sem):
        cc = lax.axis_index("c")
        ss = lax.axis_index("s")
        ebase = cc * EH + ss * EWS
        row0 = ss * RPS
        zeros16 = jnp.zeros((16,), jnp.float32)

        def zb(i, carry):
            r = i // (D // 16)
            col = (i % (D // 16)) * 16
            zbuf[r, pl.ds(col, 16)] = zeros16
            return carry
        lax.fori_loop(0, RZ * (D // 16), zb, 0)

        for g in range(NG):
            for t in range(RPS // RZ):
                pltpu.sync_copy(zbuf, accsh.at[pl.ds(row0 + t * RZ, RZ)])
            plsc.subcore_barrier()

            def body(i, carry):
                eoff = g * E + ebase + i * CH
                pltpu.sync_copy(src_hbm.at[pl.ds(eoff, CH)], isrc)
                pltpu.sync_copy(dst_hbm.at[pl.ds(eoff, CH)], idst)
                pltpu.async_copy(hs_hbm.at[g].at[isrc], rows, sem).wait()
                pltpu.sync_copy(rows, accsh.at[idst], add=True)
                return carry
            lax.fori_loop(0, n_full, body, 0)
            if rem:
                eoff = g * E + ebase + n_full * CH
                pltpu.sync_copy(src_hbm.at[pl.ds(eoff, rem)], isrc_t)
                pltpu.sync_copy(dst_hbm.at[pl.ds(eoff, rem)], idst_t)
                pltpu.async_copy(hs_hbm.at[g].at[isrc_t], rows_t, sem).wait()
                pltpu.sync_copy(rows_t, accsh.at[idst_t], add=True)
            plsc.subcore_barrier()
            pltpu.sync_copy(accsh.at[pl.ds(row0, RPS)],
                            part_hbm.at[cc, g, pl.ds(row0, RPS)])

    return k(hs, src_all.reshape(-1), dst_all.reshape(-1))


def _tc_dinv(degp):
    def body(degp_ref, dinv_ref):
        deg = jnp.sum(degp_ref[...], axis=0) + 1.0
        dinv_ref[...] = lax.rsqrt(jnp.maximum(deg, 1.0))

    return pl.pallas_call(
        body, out_shape=jax.ShapeDtypeStruct((NG, NP), jnp.float32)
    )(degp)


def _tc_prep(x_all, W, dinv):
    BN = 512
    grid = (NG, NP // BN)

    def body(x_ref, w_ref, dinv_ref, h_ref, hs_ref):
        h = jnp.dot(x_ref[0], w_ref[...], preferred_element_type=jnp.float32)
        h_ref[0] = h
        hs_ref[0] = h * dinv_ref[0]

    return pl.pallas_call(
        body,
        grid=grid,
        in_specs=[
            pl.BlockSpec((1, BN, D), lambda g, i: (g, i, 0)),
            pl.BlockSpec((D, D), lambda g, i: (0, 0)),
            pl.BlockSpec((1, BN, 1), lambda g, i: (g, i, 0)),
        ],
        out_specs=[
            pl.BlockSpec((1, BN, D), lambda g, i: (g, i, 0)),
            pl.BlockSpec((1, BN, D), lambda g, i: (g, i, 0)),
        ],
        out_shape=[jax.ShapeDtypeStruct((NG, NP, D), jnp.float32)] * 2,
    )(x_all, W, dinv[:, :, None])


def _tc_post(part, h, dinv, b, label_emb, Wq, bq, Wk, bk, Wv, bv, do_relu):
    BN = 256
    grid = (NP // BN,)
    scale = 1.0 / math.sqrt(D)

    def body(part_ref, h_ref, dinv_ref, b_ref, le_ref,
             wq_ref, bq_ref, wk_ref, bk_ref, wv_ref, bv_ref,
             gfe_ref, ofe_ref):
        dv = dinv_ref[...]
        conv = (part_ref[0] + part_ref[1]) * dv[:, :, None] \
            + h_ref[...] * (dv * dv)[:, :, None] \
            + b_ref[...][None, None, :]
        lq = jnp.dot(le_ref[...], wq_ref[...],
                     preferred_element_type=jnp.float32) + bq_ref[...][None, :]
        ks = [jnp.dot(conv[j], wk_ref[...],
                      preferred_element_type=jnp.float32) + bk_ref[...][None, :]
              for j in range(C)]
        vs = [jnp.dot(conv[j], wv_ref[...],
                      preferred_element_type=jnp.float32) + bv_ref[...][None, :]
              for j in range(C)]
        s = jnp.stack(
            [jnp.dot(kj, lq.T, preferred_element_type=jnp.float32)
             for kj in ks], axis=-1) * scale
        s = s - jnp.max(s, axis=-1, keepdims=True)
        p = jnp.exp(s)
        p = p / jnp.sum(p, axis=-1, keepdims=True)
        for a in range(C):
            out_a = p[:, a, 0][:, None] * vs[0]
            for jb in range(1, C):
                out_a = out_a + p[:, a, jb][:, None] * vs[jb]
            if do_relu:
                out_a = jnp.maximum(out_a, 0.0)
            gfe_ref[a] = out_a
        ofe = conv[C]
        if do_relu:
            ofe = jnp.maximum(ofe, 0.0)
        ofe_ref[...] = ofe

    return pl.pallas_call(
        body,
        grid=grid,
        in_specs=[
            pl.BlockSpec((NC, NG, BN, D), lambda i: (0, 0, i, 0)),
            pl.BlockSpec((NG, BN, D), lambda i: (0, i, 0)),
            pl.BlockSpec((NG, BN), lambda i: (0, i)),
            pl.BlockSpec((D,), lambda i: (0,)),
            pl.BlockSpec((C, D), lambda i: (0, 0)),
            pl.BlockSpec((D, D), lambda i: (0, 0)),
            pl.BlockSpec((D,), lambda i: (0,)),
            pl.BlockSpec((D, D), lambda i: (0, 0)),
            pl.BlockSpec((D,), lambda i: (0,)),
            pl.BlockSpec((D, D), lambda i: (0, 0)),
            pl.BlockSpec((D,), lambda i: (0,)),
        ],
        out_specs=[
            pl.BlockSpec((C, BN, D), lambda i: (0, i, 0)),
            pl.BlockSpec((BN, D), lambda i: (i, 0)),
        ],
        out_shape=[
            jax.ShapeDtypeStruct((C, NP, D), jnp.float32),
            jax.ShapeDtypeStruct((NP, D), jnp.float32),
        ],
    )(part, h, dinv, b, label_emb, Wq, bq, Wk, bk, Wv, bv)


def kernel(graph_feat_emb, ori_feat_emb, label_emb, edge_index_label,
           edge_index_ori, W0, b0, W1, b1, Wq, bq, Wk, bk, Wv, bv):
    pad = NP - N
    gfe0 = jnp.pad(graph_feat_emb, ((0, 0), (0, pad), (0, 0)))
    ofe0 = jnp.pad(ori_feat_emb, ((0, pad), (0, 0)))
    src_all = jnp.concatenate(
        [edge_index_label[:, 0, :], edge_index_ori[0][None]], axis=0)
    dst_all = jnp.concatenate(
        [edge_index_label[:, 1, :], edge_index_ori[1][None]], axis=0)

    degp = _sc_degrees(dst_all)
    dinv = _tc_dinv(degp)

    x = jnp.concatenate([gfe0, ofe0[None]], axis=0)
    gfe = None
    ofe = None
    for li, (W, b) in enumerate(((W0, b0), (W1, b1))):
        h, hs = _tc_prep(x, W, dinv)
        part = _sc_scatter(hs, src_all, dst_all)
        gfe, ofe = _tc_post(part, h, dinv, b, label_emb,
                            Wq, bq, Wk, bk, Wv, bv, do_relu=(li == 0))
        if li == 0:
            x = jnp.concatenate([gfe, ofe[None]], axis=0)

    return gfe[:, :N, :], ofe[:N, :]


# speedup vs baseline: 1.9096x; 1.0100x over previous
import functools
import math

import jax
import jax.numpy as jnp
from jax import lax
from jax.experimental import pallas as pl
from jax.experimental.pallas import tpu as pltpu
from jax.experimental.pallas import tpu_sc as plsc

N = 10000
E = 160000
C = 4
D = 128
NG = C + 1
NP = 10240
NC = 2
NS = 16
NW = NC * NS


def _sc_degrees(dst_all):
    EW = E // NW
    n_full = EW // 16
    rem = EW - n_full * 16
    mesh = plsc.VectorSubcoreMesh(core_axis_name="c", subcore_axis_name="s")

    @functools.partial(
        pl.kernel,
        out_type=jax.ShapeDtypeStruct((NW * NG * NP,), jnp.float32),
        mesh=mesh,
        scratch_types=[
            pltpu.VMEM((EW + 16,), jnp.int32),
            pltpu.VMEM((NP,), jnp.float32),
        ],
        compiler_params=pltpu.CompilerParams(needs_layout_passes=False),
    )
    def k(dst_hbm, degp_hbm, idx_v, acc):
        cc = lax.axis_index("c")
        ss = lax.axis_index("s")
        wid = ss * NC + cc
        base = wid * EW
        ones = jnp.full((16,), 1.0, jnp.float32)
        zeros16 = jnp.zeros((16,), jnp.float32)
        for g in range(NG):
            def zero_body(i, carry):
                acc[pl.ds(i * 16, 16)] = zeros16
                return carry
            lax.fori_loop(0, NP // 16, zero_body, 0)
            idx_v[pl.ds(n_full * 16, 16)] = jnp.zeros((16,), jnp.int32)
            pltpu.sync_copy(dst_hbm.at[pl.ds(g * E + base, EW)],
                            idx_v.at[pl.ds(0, EW)])

            def body(i, carry):
                idx = idx_v[pl.ds(i * 16, 16)]
                plsc.addupdate_scatter(acc, [idx], ones)
                return carry
            lax.fori_loop(0, n_full, body, 0)
            if rem:
                idx = idx_v[pl.ds(n_full * 16, 16)]
                mask = lax.iota(jnp.int32, 16) < rem
                plsc.addupdate_scatter(acc, [idx], ones, mask=mask)
            pltpu.sync_copy(acc, degp_hbm.at[pl.ds((wid * NG + g) * NP, NP)])

    return k(dst_all.reshape(-1)).reshape(NW, NG, NP)


def _sc_scatter(hs, src_all, dst_all):
    EH = E // NC
    EWS = EH // NS
    CH = 128
    n_full = EWS // CH
    rem = EWS - n_full * CH
    RZ = 128
    RPS = NP // NS
    mesh = plsc.VectorSubcoreMesh(core_axis_name="c", subcore_axis_name="s")

    @functools.partial(
        pl.kernel,
        out_type=jax.ShapeDtypeStruct((NC, NG, NP, D), jnp.float32),
        mesh=mesh,
        scratch_types=[
            pltpu.VMEM((CH,), jnp.int32),
            pltpu.VMEM((CH,), jnp.int32),
            pltpu.VMEM((CH, D), jnp.float32),
            pltpu.VMEM((rem,), jnp.int32),
            pltpu.VMEM((rem,), jnp.int32),
            pltpu.VMEM((rem, D), jnp.float32),
            pltpu.VMEM((RZ, D), jnp.float32),
            pltpu.VMEM_SHARED((NP, D), jnp.float32),
            pltpu.SemaphoreType.DMA,
        ],
    )
    def k(hs_hbm, src_hbm, dst_hbm, part_hbm,
          isrc, idst, rows, isrc_t, idst_t, rows_t, zbuf, accsh, sem):
        cc = lax.axis_index("c")
        ss = lax.axis_index("s")
        ebase = cc * EH + ss * EWS
        row0 = ss * RPS
        zeros16 = jnp.zeros((16,), jnp.float32)

        def zb(i, carry):
            r = i // (D // 16)
            col = (i % (D // 16)) * 16
            zbuf[r, pl.ds(col, 16)] = zeros16
            return carry
        lax.fori_loop(0, RZ * (D // 16), zb, 0)

        for g in range(NG):
            for t in range(RPS // RZ):
                pltpu.sync_copy(zbuf, accsh.at[pl.ds(row0 + t * RZ, RZ)])
            plsc.subcore_barrier()

            def body(i, carry):
                eoff = g * E + ebase + i * CH
                pltpu.sync_copy(src_hbm.at[pl.ds(eoff, CH)], isrc)
                pltpu.sync_copy(dst_hbm.at[pl.ds(eoff, CH)], idst)
                pltpu.async_copy(hs_hbm.at[g].at[isrc], rows, sem).wait()
                pltpu.sync_copy(rows, accsh.at[idst], add=True)
                return carry
            lax.fori_loop(0, n_full, body, 0)
            if rem:
                eoff = g * E + ebase + n_full * CH
                pltpu.sync_copy(src_hbm.at[pl.ds(eoff, rem)], isrc_t)
                pltpu.sync_copy(dst_hbm.at[pl.ds(eoff, rem)], idst_t)
                pltpu.async_copy(hs_hbm.at[g].at[isrc_t], rows_t, sem).wait()
                pltpu.sync_copy(rows_t, accsh.at[idst_t], add=True)
            plsc.subcore_barrier()
            pltpu.sync_copy(accsh.at[pl.ds(row0, RPS)],
                            part_hbm.at[cc, g, pl.ds(row0, RPS)])

    return k(hs, src_all.reshape(-1), dst_all.reshape(-1))


def _tc_dinv(degp):
    def body(degp_ref, dinv_ref):
        deg = jnp.sum(degp_ref[...], axis=0) + 1.0
        dinv_ref[...] = lax.rsqrt(jnp.maximum(deg, 1.0))

    return pl.pallas_call(
        body, out_shape=jax.ShapeDtypeStruct((NG, NP), jnp.float32)
    )(degp)


def _tc_prep(x_all, W, dinv):
    BN = 512
    grid = (NG, NP // BN)

    def body(x_ref, w_ref, dinv_ref, hs_ref):
        h = jnp.dot(x_ref[0], w_ref[...], preferred_element_type=jnp.float32)
        hs_ref[0] = h * dinv_ref[0]

    return pl.pallas_call(
        body,
        grid=grid,
        in_specs=[
            pl.BlockSpec((1, BN, D), lambda g, i: (g, i, 0)),
            pl.BlockSpec((D, D), lambda g, i: (0, 0)),
            pl.BlockSpec((1, BN, 1), lambda g, i: (g, i, 0)),
        ],
        out_specs=pl.BlockSpec((1, BN, D), lambda g, i: (g, i, 0)),
        out_shape=jax.ShapeDtypeStruct((NG, NP, D), jnp.float32),
    )(x_all, W, dinv[:, :, None])


def _tc_post(part, hs, dinv, b, label_emb, Wq, bq, Wk, bk, Wv, bv, do_relu):
    BN = 256
    grid = (NP // BN,)
    scale = 1.0 / math.sqrt(D)

    def body(part_ref, hs_ref, dinv_ref, b_ref, le_ref,
             wq_ref, bq_ref, wk_ref, bk_ref, wv_ref, bv_ref,
             gfe_ref, ofe_ref):
        dv = dinv_ref[...]
        conv = (part_ref[0] + part_ref[1] + hs_ref[...]) * dv[:, :, None] \
            + b_ref[...][None, None, :]
        lq = jnp.dot(le_ref[...], wq_ref[...],
                     preferred_element_type=jnp.float32) + bq_ref[...][None, :]
        ks = [jnp.dot(conv[j], wk_ref[...],
                      preferred_element_type=jnp.float32) + bk_ref[...][None, :]
              for j in range(C)]
        vs = [jnp.dot(conv[j], wv_ref[...],
                      preferred_element_type=jnp.float32) + bv_ref[...][None, :]
              for j in range(C)]
        s = jnp.stack(
            [jnp.dot(kj, lq.T, preferred_element_type=jnp.float32)
             for kj in ks], axis=-1) * scale
        s = s - jnp.max(s, axis=-1, keepdims=True)
        p = jnp.exp(s)
        p = p / jnp.sum(p, axis=-1, keepdims=True)
        for a in range(C):
            out_a = p[:, a, 0][:, None] * vs[0]
            for jb in range(1, C):
                out_a = out_a + p[:, a, jb][:, None] * vs[jb]
            if do_relu:
                out_a = jnp.maximum(out_a, 0.0)
            gfe_ref[a] = out_a
        ofe = conv[C]
        if do_relu:
            ofe = jnp.maximum(ofe, 0.0)
        ofe_ref[...] = ofe

    return pl.pallas_call(
        body,
        grid=grid,
        in_specs=[
            pl.BlockSpec((NC, NG, BN, D), lambda i: (0, 0, i, 0)),
            pl.BlockSpec((NG, BN, D), lambda i: (0, i, 0)),
            pl.BlockSpec((NG, BN), lambda i: (0, i)),
            pl.BlockSpec((D,), lambda i: (0,)),
            pl.BlockSpec((C, D), lambda i: (0, 0)),
            pl.BlockSpec((D, D), lambda i: (0, 0)),
            pl.BlockSpec((D,), lambda i: (0,)),
            pl.BlockSpec((D, D), lambda i: (0, 0)),
            pl.BlockSpec((D,), lambda i: (0,)),
            pl.BlockSpec((D, D), lambda i: (0, 0)),
            pl.BlockSpec((D,), lambda i: (0,)),
        ],
        out_specs=[
            pl.BlockSpec((C, BN, D), lambda i: (0, i, 0)),
            pl.BlockSpec((BN, D), lambda i: (i, 0)),
        ],
        out_shape=[
            jax.ShapeDtypeStruct((C, NP, D), jnp.float32),
            jax.ShapeDtypeStruct((NP, D), jnp.float32),
        ],
    )(part, hs, dinv, b, label_emb, Wq, bq, Wk, bk, Wv, bv)


def kernel(graph_feat_emb, ori_feat_emb, label_emb, edge_index_label,
           edge_index_ori, W0, b0, W1, b1, Wq, bq, Wk, bk, Wv, bv):
    pad = NP - N
    gfe0 = jnp.pad(graph_feat_emb, ((0, 0), (0, pad), (0, 0)))
    ofe0 = jnp.pad(ori_feat_emb, ((0, pad), (0, 0)))
    src_all = jnp.concatenate(
        [edge_index_label[:, 0, :], edge_index_ori[0][None]], axis=0)
    dst_all = jnp.concatenate(
        [edge_index_label[:, 1, :], edge_index_ori[1][None]], axis=0)

    degp = _sc_degrees(dst_all)
    dinv = _tc_dinv(degp)

    x = jnp.concatenate([gfe0, ofe0[None]], axis=0)
    gfe = None
    ofe = None
    for li, (W, b) in enumerate(((W0, b0), (W1, b1))):
        hs = _tc_prep(x, W, dinv)
        part = _sc_scatter(hs, src_all, dst_all)
        gfe, ofe = _tc_post(part, hs, dinv, b, label_emb,
                            Wq, bq, Wk, bk, Wv, bv, do_relu=(li == 0))
        if li == 0:
            x = jnp.concatenate([gfe, ofe[None]], axis=0)

    return gfe[:, :N, :], ofe[:N, :]
